# baseline (speedup 1.0000x reference)
import functools

import jax
import jax.numpy as jnp
from jax import lax
from jax.experimental import pallas as pl
from jax.experimental.pallas import tpu as pltpu
from jax.experimental.pallas import tpu_sc as plsc

N = 10000
E = 320000
D = 128
H = 64
NREL = 2
NC = 2
NS = 16

EROWS = 2560
EBLK = EROWS // 8
EP = EROWS * 128
ACC_ROWS = 2 * N + 96
ZROWS = ACC_ROWS // NS
WOUT = 1256

_mesh = plsc.VectorSubcoreMesh(core_axis_name="c", subcore_axis_name="s")


def _enc_body(x_ref, w_ref, b_ref, dst_ref, typ_ref, out_ref, dstc_ref):
    h = jnp.dot(x_ref[...], w_ref[...], preferred_element_type=jnp.float32)
    h = h + b_ref[...]
    h = jnp.where(h >= 0, h, 0.01 * h)
    out_ref[0] = h[:, :H]
    out_ref[1] = h[:, H:]
    dstc_ref[...] = dst_ref[...] + N * typ_ref[...]


def _encoder(x, W_in, b_in, dst2, typ2):
    blk = 1000
    eblk = EROWS // 10
    return pl.pallas_call(
        _enc_body,
        grid=(N // blk,),
        in_specs=[
            pl.BlockSpec((blk, D), lambda i: (i, 0)),
            pl.BlockSpec((D, D), lambda i: (0, 0)),
            pl.BlockSpec((1, D), lambda i: (0, 0)),
            pl.BlockSpec((eblk, 128), lambda i: (i, 0)),
            pl.BlockSpec((eblk, 128), lambda i: (i, 0)),
        ],
        out_specs=[
            pl.BlockSpec((2, blk, H), lambda i: (0, i, 0)),
            pl.BlockSpec((eblk, 128), lambda i: (i, 0)),
        ],
        out_shape=[
            jax.ShapeDtypeStruct((2, N, H), jnp.float32),
            jax.ShapeDtypeStruct((EROWS, 128), jnp.int32),
        ],
    )(x, W_in, b_in, dst2, typ2)


def _rgcn_acc(hs_ref, sums_ref, cnt_ref, root_ref, w_ref, b_ref):
    acc = jnp.dot(hs_ref[0], root_ref[:H, :], preferred_element_type=jnp.float32)
    acc += jnp.dot(hs_ref[1], root_ref[H:, :], preferred_element_type=jnp.float32)
    acc += b_ref[...]
    for r in range(NREL):
        cnt = cnt_ref[0, 0, r] + cnt_ref[0, 1, r]
        inv = (1.0 / jnp.maximum(cnt, 1.0))[:, None]
        acc += jnp.dot(sums_ref[0, r] * inv, w_ref[r, :H, :],
                       preferred_element_type=jnp.float32)
        acc += jnp.dot(sums_ref[1, r] * inv, w_ref[r, H:, :],
                       preferred_element_type=jnp.float32)
    return jnp.where(acc >= 0, acc, 0.01 * acc)


def _combine_body(hs_ref, sums_ref, cnt_ref, root_ref, w_ref, b_ref, out_ref):
    h = _rgcn_acc(hs_ref, sums_ref, cnt_ref, root_ref, w_ref, b_ref)
    out_ref[0] = h[:, :H]
    out_ref[1] = h[:, H:]


def _combine_final_body(hs_ref, sums_ref, cnt_ref, root_ref, w_ref, b_ref,
                        wc_ref, bc_ref, out_ref):
    h = _rgcn_acc(hs_ref, sums_ref, cnt_ref, root_ref, w_ref, b_ref)
    out_ref[...] = (jnp.dot(h, wc_ref[...], preferred_element_type=jnp.float32)
                    + bc_ref[...])


_COMBINE_SPECS = [
    pl.BlockSpec((2, 1000, H), lambda i: (0, i, 0)),
    pl.BlockSpec((2, NREL, 1000, H), lambda i: (0, 0, i, 0)),
    pl.BlockSpec((1, 2, NREL, 1000), lambda i: (i, 0, 0, 0)),
    pl.BlockSpec((D, D), lambda i: (0, 0)),
    pl.BlockSpec((NREL, D, D), lambda i: (0, 0, 0)),
    pl.BlockSpec((1, D), lambda i: (0, 0)),
]


def _combine(hs, sums4, cnt4, root, W, bias):
    return pl.pallas_call(
        _combine_body,
        grid=(10,),
        in_specs=_COMBINE_SPECS,
        out_specs=pl.BlockSpec((2, 1000, H), lambda i: (0, i, 0)),
        out_shape=jax.ShapeDtypeStruct((2, N, H), jnp.float32),
    )(hs, sums4, cnt4, root, W, bias)


def _combine_final(hs, sums4, cnt4, root, W, bias, Wc, bc):
    return pl.pallas_call(
        _combine_final_body,
        grid=(10,),
        in_specs=_COMBINE_SPECS + [
            pl.BlockSpec((D, D), lambda i: (0, 0)),
            pl.BlockSpec((1, D), lambda i: (0, 0)),
        ],
        out_specs=pl.BlockSpec((1000, D), lambda i: (i, 0)),
        out_shape=jax.ShapeDtypeStruct((N, D), jnp.float32),
    )(hs, sums4, cnt4, root, W, bias, Wc, bc)


def _writeout(src_s, out_hbm, c, s):
    @pl.when(s < NS - 1)
    def _():
        pltpu.sync_copy(src_s.at[pl.ds(s * WOUT, WOUT)],
                        out_hbm.at[c, pl.ds(s * WOUT, WOUT)])

    @pl.when(s == NS - 1)
    def _():
        off = (NS - 1) * WOUT
        rem = 2 * N - off
        pltpu.sync_copy(src_s.at[pl.ds(off, rem)],
                        out_hbm.at[c, pl.ds(off, rem)])


@functools.partial(
    pl.kernel,
    out_type=(jax.ShapeDtypeStruct((2, 2 * N, H), jnp.float32),
              jax.ShapeDtypeStruct((2, 2 * N, 8), jnp.float32)),
    mesh=_mesh,
    compiler_params=pltpu.CompilerParams(use_tc_tiling_on_sc=False),
    scratch_types=[
        pltpu.VMEM((2, 8, 128), jnp.int32),
        pltpu.VMEM((2, 8, 128), jnp.int32),
        pltpu.VMEM((4, 128, H), jnp.float32),
        pltpu.VMEM((128, 8), jnp.float32),
        pltpu.SemaphoreType.DMA,
        pltpu.SemaphoreType.DMA,
        pltpu.SemaphoreType.DMA,
        pltpu.VMEM_SHARED((ACC_ROWS, H), jnp.float32),
        pltpu.VMEM_SHARED((ACC_ROWS, 8), jnp.float32),
    ],
)
def _sc_segsum_cnt(hs_hbm, src_hbm, dstc_hbm, zeros_hbm, zeros8_hbm, ones_hbm,
                   out_hbm, cnt_hbm, gidx_v, sidx_v, rows_v, ones_v,
                   gsem, ssem, csem, acc_s, cnt_s):
    c = lax.axis_index("c")
    s = lax.axis_index("s")
    pltpu.sync_copy(zeros_hbm.at[pl.ds(s * ZROWS, ZROWS)],
                    acc_s.at[pl.ds(s * ZROWS, ZROWS)])
    pltpu.sync_copy(zeros8_hbm.at[pl.ds(s * ZROWS, ZROWS)],
                    cnt_s.at[pl.ds(s * ZROWS, ZROWS)])
    pltpu.sync_copy(ones_hbm, ones_v)
    plsc.subcore_barrier()
    bpt = EBLK // NS
    nsup = bpt // 2

    def body(j, carry):
        rbase = s * bpt + j * 2
        do_cnt = lax.rem(j, 2) == c
        pltpu.sync_copy(src_hbm.at[pl.ds(rbase, 2)], gidx_v)
        pltpu.sync_copy(dstc_hbm.at[pl.ds(rbase, 2)], sidx_v)
        gd = [pltpu.async_copy(hs_hbm.at[c].at[gidx_v.at[m // 8, m % 8]],
                               rows_v.at[m % 4], gsem) for m in range(4)]
        sd, cd = [], []
        for m in range(16):
            gd[m].wait()
            sd.append(pltpu.async_copy(rows_v.at[m % 4],
                                       acc_s.at[sidx_v.at[m // 8, m % 8]],
                                       ssem, add=True))

            @pl.when(do_cnt)
            def _(m=m):
                cd.append(pltpu.async_copy(ones_v,
                                           cnt_s.at[sidx_v.at[m // 8, m % 8]],
                                           csem, add=True))
                if m >= 8:
                    cd[m - 8].wait()

            n = m + 4
            if n < 16:
                sd[m].wait()
                gd.append(pltpu.async_copy(
                    hs_hbm.at[c].at[gidx_v.at[n // 8, n % 8]],
                    rows_v.at[n % 4], gsem))
        for d in sd[12:]:
            d.wait()

        @pl.when(do_cnt)
        def _():
            for d in cd[8:]:
                d.wait()

        return carry

    lax.fori_loop(0, nsup, body, 0)
    plsc.subcore_barrier()
    _writeout(acc_s, out_hbm, c, s)
    _writeout(cnt_s, cnt_hbm, c, s)


@functools.partial(
    pl.kernel,
    out_type=jax.ShapeDtypeStruct((2, 2 * N, H), jnp.float32),
    mesh=_mesh,
    compiler_params=pltpu.CompilerParams(use_tc_tiling_on_sc=False),
    scratch_types=[
        pltpu.VMEM((4, 8, 128), jnp.int32),
        pltpu.VMEM((4, 8, 128), jnp.int32),
        pltpu.VMEM((5, 128, H), jnp.float32),
        pltpu.SemaphoreType.DMA,
        pltpu.SemaphoreType.DMA,
        pltpu.VMEM_SHARED((ACC_ROWS, H), jnp.float32),
    ],
)
def _sc_segsum(hs_hbm, src_hbm, dstc_hbm, zeros_hbm, out_hbm,
               gidx_v, sidx_v, rows_v, gsem, ssem, acc_s):
    c = lax.axis_index("c")
    s = lax.axis_index("s")
    pltpu.sync_copy(zeros_hbm.at[pl.ds(s * ZROWS, ZROWS)],
                    acc_s.at[pl.ds(s * ZROWS, ZROWS)])
    plsc.subcore_barrier()
    bpt = EBLK // NS
    nsup = bpt // 4

    def body(j, carry):
        rbase = s * bpt + j * 4
        pltpu.sync_copy(src_hbm.at[pl.ds(rbase, 4)], gidx_v)
        pltpu.sync_copy(dstc_hbm.at[pl.ds(rbase, 4)], sidx_v)
        gd = [pltpu.async_copy(hs_hbm.at[c].at[gidx_v.at[m // 8, m % 8]],
                               rows_v.at[m % 5], gsem) for m in range(5)]
        sd = []
        for m in range(32):
            gd[m].wait()
            sd.append(pltpu.async_copy(rows_v.at[m % 5],
                                       acc_s.at[sidx_v.at[m // 8, m % 8]],
                                       ssem, add=True))
            n = m + 5
            if n < 32:
                sd[m].wait()
                gd.append(pltpu.async_copy(
                    hs_hbm.at[c].at[gidx_v.at[n // 8, n % 8]],
                    rows_v.at[n % 5], gsem))
        for d in sd[27:]:
            d.wait()
        return carry

    lax.fori_loop(0, nsup, body, 0)
    plsc.subcore_barrier()
    _writeout(acc_s, out_hbm, c, s)


def kernel(x, edge_index, edge_type, W_in, b_in, W1, root1, bias1,
           W2, root2, bias2, Wc, bc):
    src = edge_index[0]
    dst = edge_index[1]
    pad = EP - E
    ar = jnp.arange(pad, dtype=jnp.int32)
    src2 = jnp.concatenate([src, ar % N]).reshape(EROWS, 128)
    dst2 = jnp.concatenate([dst, 2 * N + (ar % 8)]).reshape(EROWS, 128)
    typ2 = jnp.concatenate([edge_type, jnp.zeros((pad,), jnp.int32)]
                           ).reshape(EROWS, 128)
    src3 = src2.reshape(EBLK, 8, 128)

    ones8 = jnp.ones((128, 8), jnp.float32)
    zeros8 = jnp.zeros((ACC_ROWS, 8), jnp.float32)
    zeros64 = jnp.zeros((ACC_ROWS, H), jnp.float32)

    hs1, dstc2 = _encoder(x, W_in, b_in[None, :], dst2, typ2)
    dstc3 = dstc2.reshape(EBLK, 8, 128)
    sums1, cnt = _sc_segsum_cnt(hs1, src3, dstc3, zeros64, zeros8, ones8)
    sums1 = sums1.reshape(2, NREL, N, H)
    cnt4 = cnt[:, :, 0].reshape(2, NREL, 10, 1000).transpose(2, 0, 1, 3)
    hs2 = _combine(hs1, sums1, cnt4, root1, W1, bias1[None, :])
    sums2 = _sc_segsum(hs2, src3, dstc3, zeros64).reshape(2, NREL, N, H)
    return _combine_final(hs2, sums2, cnt4, root2, W2, bias2[None, :],
                          Wc, bc[None, :])


# speedup vs baseline: 1.0023x; 1.0023x over previous
import functools

import jax
import jax.numpy as jnp
from jax import lax
from jax.experimental import pallas as pl
from jax.experimental.pallas import tpu as pltpu
from jax.experimental.pallas import tpu_sc as plsc

N = 10000
E = 320000
D = 128
H = 64
NREL = 2
NC = 2
NS = 16

EROWS = 2560
EBLK = EROWS // 8
EP = EROWS * 128
ACC_ROWS = 2 * N + 96
ZROWS = ACC_ROWS // NS
WOUT = 1256

_mesh = plsc.VectorSubcoreMesh(core_axis_name="c", subcore_axis_name="s")


def _enc_body(x_ref, w_ref, b_ref, dst_ref, typ_ref, out_ref, dstc_ref):
    h = jnp.dot(x_ref[...], w_ref[...], preferred_element_type=jnp.float32)
    h = h + b_ref[...]
    h = jnp.where(h >= 0, h, 0.01 * h)
    out_ref[0] = h[:, :H]
    out_ref[1] = h[:, H:]
    dstc_ref[...] = dst_ref[...] + N * typ_ref[...]


def _encoder(x, W_in, b_in, dst2, typ2):
    blk = 1000
    eblk = EROWS // 10
    return pl.pallas_call(
        _enc_body,
        grid=(N // blk,),
        in_specs=[
            pl.BlockSpec((blk, D), lambda i: (i, 0)),
            pl.BlockSpec((D, D), lambda i: (0, 0)),
            pl.BlockSpec((1, D), lambda i: (0, 0)),
            pl.BlockSpec((eblk, 128), lambda i: (i, 0)),
            pl.BlockSpec((eblk, 128), lambda i: (i, 0)),
        ],
        out_specs=[
            pl.BlockSpec((2, blk, H), lambda i: (0, i, 0)),
            pl.BlockSpec((eblk, 128), lambda i: (i, 0)),
        ],
        out_shape=[
            jax.ShapeDtypeStruct((2, N, H), jnp.float32),
            jax.ShapeDtypeStruct((EROWS, 128), jnp.int32),
        ],
    )(x, W_in, b_in, dst2, typ2)


def _rgcn_acc(hs_ref, sums_ref, cnt_ref, root_ref, w_ref, b_ref):
    acc = jnp.dot(hs_ref[0], root_ref[:H, :], preferred_element_type=jnp.float32)
    acc += jnp.dot(hs_ref[1], root_ref[H:, :], preferred_element_type=jnp.float32)
    acc += b_ref[...]
    for r in range(NREL):
        cnt = cnt_ref[0, r]
        inv = (1.0 / jnp.maximum(cnt, 1.0))[:, None]
        acc += jnp.dot(sums_ref[0, r] * inv, w_ref[r, :H, :],
                       preferred_element_type=jnp.float32)
        acc += jnp.dot(sums_ref[1, r] * inv, w_ref[r, H:, :],
                       preferred_element_type=jnp.float32)
    return jnp.where(acc >= 0, acc, 0.01 * acc)


def _combine_body(hs_ref, sums_ref, cnt_ref, root_ref, w_ref, b_ref, out_ref):
    h = _rgcn_acc(hs_ref, sums_ref, cnt_ref, root_ref, w_ref, b_ref)
    out_ref[0] = h[:, :H]
    out_ref[1] = h[:, H:]


def _combine_final_body(hs_ref, sums_ref, cnt_ref, root_ref, w_ref, b_ref,
                        wc_ref, bc_ref, out_ref):
    h = _rgcn_acc(hs_ref, sums_ref, cnt_ref, root_ref, w_ref, b_ref)
    out_ref[...] = (jnp.dot(h, wc_ref[...], preferred_element_type=jnp.float32)
                    + bc_ref[...])


_COMBINE_SPECS = [
    pl.BlockSpec((2, 1000, H), lambda i: (0, i, 0)),
    pl.BlockSpec((2, NREL, 1000, H), lambda i: (0, 0, i, 0)),
    pl.BlockSpec((1, NREL, 1000), lambda i: (i, 0, 0)),
    pl.BlockSpec((D, D), lambda i: (0, 0)),
    pl.BlockSpec((NREL, D, D), lambda i: (0, 0, 0)),
    pl.BlockSpec((1, D), lambda i: (0, 0)),
]


def _combine(hs, sums4, cnt4, root, W, bias):
    return pl.pallas_call(
        _combine_body,
        grid=(10,),
        in_specs=_COMBINE_SPECS,
        out_specs=pl.BlockSpec((2, 1000, H), lambda i: (0, i, 0)),
        out_shape=jax.ShapeDtypeStruct((2, N, H), jnp.float32),
    )(hs, sums4, cnt4, root, W, bias)


def _combine_final(hs, sums4, cnt4, root, W, bias, Wc, bc):
    return pl.pallas_call(
        _combine_final_body,
        grid=(10,),
        in_specs=_COMBINE_SPECS + [
            pl.BlockSpec((D, D), lambda i: (0, 0)),
            pl.BlockSpec((1, D), lambda i: (0, 0)),
        ],
        out_specs=pl.BlockSpec((1000, D), lambda i: (i, 0)),
        out_shape=jax.ShapeDtypeStruct((N, D), jnp.float32),
    )(hs, sums4, cnt4, root, W, bias, Wc, bc)


def _writeout(src_s, out_hbm, c, s):
    @pl.when(s < NS - 1)
    def _():
        pltpu.sync_copy(src_s.at[pl.ds(s * WOUT, WOUT)],
                        out_hbm.at[c, pl.ds(s * WOUT, WOUT)])

    @pl.when(s == NS - 1)
    def _():
        off = (NS - 1) * WOUT
        rem = 2 * N - off
        pltpu.sync_copy(src_s.at[pl.ds(off, rem)],
                        out_hbm.at[c, pl.ds(off, rem)])


@functools.partial(
    pl.kernel,
    out_type=(jax.ShapeDtypeStruct((2, 2 * N, H), jnp.float32),
              jax.ShapeDtypeStruct((2, 2 * N, 8), jnp.float32)),
    mesh=_mesh,
    compiler_params=pltpu.CompilerParams(use_tc_tiling_on_sc=False),
    scratch_types=[
        pltpu.VMEM((2, 8, 128), jnp.int32),
        pltpu.VMEM((2, 8, 128), jnp.int32),
        pltpu.VMEM((4, 128, H), jnp.float32),
        pltpu.VMEM((128, 8), jnp.float32),
        pltpu.SemaphoreType.DMA,
        pltpu.SemaphoreType.DMA,
        pltpu.SemaphoreType.DMA,
        pltpu.VMEM_SHARED((ACC_ROWS, H), jnp.float32),
        pltpu.VMEM_SHARED((ACC_ROWS, 8), jnp.float32),
    ],
)
def _sc_segsum_cnt(hs_hbm, src_hbm, dstc_hbm, zeros_hbm, zeros8_hbm, ones_hbm,
                   out_hbm, cnt_hbm, gidx_v, sidx_v, rows_v, ones_v,
                   gsem, ssem, csem, acc_s, cnt_s):
    c = lax.axis_index("c")
    s = lax.axis_index("s")
    pltpu.sync_copy(zeros_hbm.at[pl.ds(s * ZROWS, ZROWS)],
                    acc_s.at[pl.ds(s * ZROWS, ZROWS)])
    pltpu.sync_copy(zeros8_hbm.at[pl.ds(s * ZROWS, ZROWS)],
                    cnt_s.at[pl.ds(s * ZROWS, ZROWS)])
    pltpu.sync_copy(ones_hbm, ones_v)
    plsc.subcore_barrier()
    bpt = EBLK // NS
    nsup = bpt // 2

    def body(j, carry):
        rbase = s * bpt + j * 2
        pltpu.sync_copy(src_hbm.at[pl.ds(rbase, 2)], gidx_v)
        pltpu.sync_copy(dstc_hbm.at[pl.ds(rbase, 2)], sidx_v)
        gd = [pltpu.async_copy(hs_hbm.at[c].at[gidx_v.at[m // 8, m % 8]],
                               rows_v.at[m % 4], gsem) for m in range(4)]
        sd, cd = [], []
        for m in range(16):
            gd[m].wait()
            sd.append(pltpu.async_copy(rows_v.at[m % 4],
                                       acc_s.at[sidx_v.at[m // 8, m % 8]],
                                       ssem, add=True))
            cd.append(pltpu.async_copy(ones_v,
                                       cnt_s.at[sidx_v.at[m // 8, m % 8]],
                                       csem, add=True))
            if m >= 8:
                cd[m - 8].wait()
            n = m + 4
            if n < 16:
                sd[m].wait()
                gd.append(pltpu.async_copy(
                    hs_hbm.at[c].at[gidx_v.at[n // 8, n % 8]],
                    rows_v.at[n % 4], gsem))
        for d in sd[12:]:
            d.wait()
        for d in cd[8:]:
            d.wait()
        return carry

    lax.fori_loop(0, nsup, body, 0)
    plsc.subcore_barrier()
    _writeout(acc_s, out_hbm, c, s)
    _writeout(cnt_s, cnt_hbm, c, s)


@functools.partial(
    pl.kernel,
    out_type=jax.ShapeDtypeStruct((2, 2 * N, H), jnp.float32),
    mesh=_mesh,
    compiler_params=pltpu.CompilerParams(use_tc_tiling_on_sc=False),
    scratch_types=[
        pltpu.VMEM((4, 8, 128), jnp.int32),
        pltpu.VMEM((4, 8, 128), jnp.int32),
        pltpu.VMEM((5, 128, H), jnp.float32),
        pltpu.SemaphoreType.DMA,
        pltpu.SemaphoreType.DMA,
        pltpu.VMEM_SHARED((ACC_ROWS, H), jnp.float32),
    ],
)
def _sc_segsum(hs_hbm, src_hbm, dstc_hbm, zeros_hbm, out_hbm,
               gidx_v, sidx_v, rows_v, gsem, ssem, acc_s):
    c = lax.axis_index("c")
    s = lax.axis_index("s")
    pltpu.sync_copy(zeros_hbm.at[pl.ds(s * ZROWS, ZROWS)],
                    acc_s.at[pl.ds(s * ZROWS, ZROWS)])
    plsc.subcore_barrier()
    bpt = EBLK // NS
    nsup = bpt // 4

    def body(j, carry):
        rbase = s * bpt + j * 4
        pltpu.sync_copy(src_hbm.at[pl.ds(rbase, 4)], gidx_v)
        pltpu.sync_copy(dstc_hbm.at[pl.ds(rbase, 4)], sidx_v)
        gd = [pltpu.async_copy(hs_hbm.at[c].at[gidx_v.at[m // 8, m % 8]],
                               rows_v.at[m % 5], gsem) for m in range(5)]
        sd = []
        for m in range(32):
            gd[m].wait()
            sd.append(pltpu.async_copy(rows_v.at[m % 5],
                                       acc_s.at[sidx_v.at[m // 8, m % 8]],
                                       ssem, add=True))
            n = m + 5
            if n < 32:
                sd[m].wait()
                gd.append(pltpu.async_copy(
                    hs_hbm.at[c].at[gidx_v.at[n // 8, n % 8]],
                    rows_v.at[n % 5], gsem))
        for d in sd[27:]:
            d.wait()
        return carry

    lax.fori_loop(0, nsup, body, 0)
    plsc.subcore_barrier()
    _writeout(acc_s, out_hbm, c, s)


def kernel(x, edge_index, edge_type, W_in, b_in, W1, root1, bias1,
           W2, root2, bias2, Wc, bc):
    src = edge_index[0]
    dst = edge_index[1]
    pad = EP - E
    ar = jnp.arange(pad, dtype=jnp.int32)
    src2 = jnp.concatenate([src, ar % N]).reshape(EROWS, 128)
    dst2 = jnp.concatenate([dst, 2 * N + (ar % 8)]).reshape(EROWS, 128)
    typ2 = jnp.concatenate([edge_type, jnp.zeros((pad,), jnp.int32)]
                           ).reshape(EROWS, 128)
    src3 = src2.reshape(EBLK, 8, 128)

    ones8 = jnp.ones((128, 8), jnp.float32)
    zeros8 = jnp.zeros((ACC_ROWS, 8), jnp.float32)
    zeros64 = jnp.zeros((ACC_ROWS, H), jnp.float32)

    hs1, dstc2 = _encoder(x, W_in, b_in[None, :], dst2, typ2)
    dstc3 = dstc2.reshape(EBLK, 8, 128)
    sums1, cnt = _sc_segsum_cnt(hs1, src3, dstc3, zeros64, zeros8, ones8)
    sums1 = sums1.reshape(2, NREL, N, H)
    cnt4 = cnt[0, :, 0].reshape(NREL, 10, 1000).transpose(1, 0, 2)
    hs2 = _combine(hs1, sums1, cnt4, root1, W1, bias1[None, :])
    sums2 = _sc_segsum(hs2, src3, dstc3, zeros64).reshape(2, NREL, N, H)
    return _combine_final(hs2, sums2, cnt4, root2, W2, bias2[None, :],
                          Wc, bc[None, :])
